# baseline (speedup 1.0000x reference)
import functools

import jax
import jax.numpy as jnp
from jax import lax
from jax.experimental import pallas as pl
from jax.experimental.pallas import tpu as pltpu
from jax.experimental.pallas import tpu_sc as plsc

EMBED_DIM = 32
BATCH = 16384
LANES = 16
CHUNK = 128
N_CHUNKS_PER_WORKER = 4


def _num_workers():
    info = plsc.get_sparse_core_info()
    return info.num_cores, info.num_subcores


def _tower_body(user_hbm, item_hbm, uid_hbm, iid_hbm, out_hbm,
                uidx, iidx, urows, irows, oacc, sem):
    nc, _ = _num_workers()
    wid = lax.axis_index("s") * nc + lax.axis_index("c")
    b_per_w = N_CHUNKS_PER_WORKER * CHUNK
    base = wid * b_per_w

    for j in range(N_CHUNKS_PER_WORKER):
        pltpu.sync_copy(uid_hbm.at[pl.ds(base + j * CHUNK, CHUNK)], uidx.at[j])
        pltpu.sync_copy(iid_hbm.at[pl.ds(base + j * CHUNK, CHUNK)], iidx.at[j])

    copies = []
    for j in range(N_CHUNKS_PER_WORKER):
        copies.append(pltpu.async_copy(
            user_hbm.at[uidx.at[j]], urows.at[pl.ds(j * CHUNK, CHUNK)], sem))
        copies.append(pltpu.async_copy(
            item_hbm.at[iidx.at[j]], irows.at[pl.ds(j * CHUNK, CHUNK)], sem))
    for c in copies:
        c.wait()

    iota16 = lax.iota(jnp.int32, LANES)
    n_groups = b_per_w // LANES

    def group_body(g, _):
        row_idx = g * LANES + iota16
        acc = jnp.zeros((LANES,), jnp.float32)
        for d in range(EMBED_DIM):
            col = jnp.full((LANES,), d, jnp.int32)
            u = plsc.load_gather(urows, [row_idx, col])
            v = plsc.load_gather(irows, [row_idx, col])
            acc = acc + u * v
        sig = 1.0 / (1.0 + jnp.exp(-acc))
        oacc[pl.ds(g * LANES, LANES)] = sig
        return _

    lax.fori_loop(0, n_groups, group_body, None)

    pltpu.sync_copy(oacc, out_hbm.at[pl.ds(base, b_per_w)])


@jax.jit
def _two_tower(user_table, item_table, user_ids, item_ids):
    b_per_w = N_CHUNKS_PER_WORKER * CHUNK
    mesh = plsc.VectorSubcoreMesh(core_axis_name="c", subcore_axis_name="s")
    run = functools.partial(
        pl.kernel,
        mesh=mesh,
        out_type=jax.ShapeDtypeStruct((BATCH,), jnp.float32),
        scratch_types=[
            pltpu.VMEM((N_CHUNKS_PER_WORKER, CHUNK), jnp.int32),
            pltpu.VMEM((N_CHUNKS_PER_WORKER, CHUNK), jnp.int32),
            pltpu.VMEM((b_per_w, EMBED_DIM), jnp.float32),
            pltpu.VMEM((b_per_w, EMBED_DIM), jnp.float32),
            pltpu.VMEM((b_per_w,), jnp.float32),
            pltpu.SemaphoreType.DMA,
        ],
        compiler_params=pltpu.CompilerParams(
            needs_layout_passes=False, use_tc_tiling_on_sc=False),
    )(_tower_body)
    return run(user_table, item_table, user_ids, item_ids)


def kernel(user_table, item_table, user_ID_list, item_ID_list):
    return _two_tower(user_table, item_table,
                      user_ID_list.astype(jnp.int32),
                      item_ID_list.astype(jnp.int32))


# speedup vs baseline: 3.2957x; 3.2957x over previous
import functools

import jax
import jax.numpy as jnp
from jax import lax
from jax.experimental import pallas as pl
from jax.experimental.pallas import tpu as pltpu
from jax.experimental.pallas import tpu_sc as plsc

EMBED_DIM = 32
BATCH = 16384
LANES = 16
B_PER_W = 512
N_GROUPS = B_PER_W // LANES
NBUF = 8


def _num_cores():
    return plsc.get_sparse_core_info().num_cores


def _tower_body(ut_hbm, it_hbm, uid_hbm, iid_hbm, out_hbm,
                uids_v, iids_v, ubufs, vbufs, ucolsT, icolsT, out_v,
                sem_u, sem_i):
    nc = _num_cores()
    wid = lax.axis_index("s") * nc + lax.axis_index("c")
    base = wid * B_PER_W

    pltpu.sync_copy(uid_hbm.at[pl.ds(base, B_PER_W)], uids_v)
    pltpu.sync_copy(iid_hbm.at[pl.ds(base, B_PER_W)], iids_v)

    iota16 = lax.iota(jnp.int32, LANES)

    def fetch_group(g, _):
        uv = uids_v[pl.ds(g * LANES, LANES)]
        iv = iids_v[pl.ds(g * LANES, LANES)]
        for half in range(2):
            copies = []
            ids_u, ids_i = [], []
            for b in range(NBUF):
                idu = uv[half * NBUF + b]
                idi = iv[half * NBUF + b]
                ids_u.append(idu)
                ids_i.append(idi)
                cu = pl.multiple_of(jnp.bitwise_and(idu, -128), 128)
                ci = pl.multiple_of(jnp.bitwise_and(idi, -128), 128)
                copies.append(pltpu.async_copy(
                    ut_hbm.at[:, pl.ds(cu, 128)], ubufs.at[b], sem_u))
                copies.append(pltpu.async_copy(
                    it_hbm.at[:, pl.ds(ci, 128)], vbufs.at[b], sem_i))
            for cp in copies:
                cp.wait()
            for b in range(NBUF):
                slot = g * LANES + half * NBUF + b
                cu = jnp.full((LANES,), jnp.bitwise_and(ids_u[b], 127),
                              jnp.int32)
                ci = jnp.full((LANES,), jnp.bitwise_and(ids_i[b], 127),
                              jnp.int32)
                for h in range(2):
                    rows = h * LANES + iota16
                    dpos = (h * LANES + iota16) * B_PER_W + slot
                    u = plsc.load_gather(ubufs.at[b], [rows, cu])
                    v = plsc.load_gather(vbufs.at[b], [rows, ci])
                    plsc.store_scatter(ucolsT, [dpos], u)
                    plsc.store_scatter(icolsT, [dpos], v)
        return _

    lax.fori_loop(0, N_GROUPS, fetch_group, None)

    def compute(g, _):
        acc = jnp.zeros((LANES,), jnp.float32)
        for d in range(EMBED_DIM):
            u = ucolsT[pl.ds(d * B_PER_W + g * LANES, LANES)]
            v = icolsT[pl.ds(d * B_PER_W + g * LANES, LANES)]
            acc = acc + u * v
        sig = 1.0 / (1.0 + jnp.exp(-acc))
        out_v[pl.ds(g * LANES, LANES)] = sig
        return _

    lax.fori_loop(0, N_GROUPS, compute, None)

    pltpu.sync_copy(out_v, out_hbm.at[pl.ds(base, B_PER_W)])


@jax.jit
def _two_tower(user_table, item_table, user_ids, item_ids):
    mesh = plsc.VectorSubcoreMesh(core_axis_name="c", subcore_axis_name="s")
    run = functools.partial(
        pl.kernel,
        mesh=mesh,
        out_type=jax.ShapeDtypeStruct((BATCH,), jnp.float32),
        scratch_types=[
            pltpu.VMEM((B_PER_W,), jnp.int32),
            pltpu.VMEM((B_PER_W,), jnp.int32),
            pltpu.VMEM((NBUF, EMBED_DIM, 128), jnp.float32),
            pltpu.VMEM((NBUF, EMBED_DIM, 128), jnp.float32),
            pltpu.VMEM((B_PER_W * EMBED_DIM,), jnp.float32),
            pltpu.VMEM((B_PER_W * EMBED_DIM,), jnp.float32),
            pltpu.VMEM((B_PER_W,), jnp.float32),
            pltpu.SemaphoreType.DMA,
            pltpu.SemaphoreType.DMA,
        ],
        compiler_params=pltpu.CompilerParams(
            needs_layout_passes=False, use_tc_tiling_on_sc=True,
            disable_bounds_checks=True),
    )(_tower_body)
    return run(user_table.T, item_table.T, user_ids, item_ids)


def kernel(user_table, item_table, user_ID_list, item_ID_list):
    return _two_tower(user_table, item_table,
                      user_ID_list.astype(jnp.int32),
                      item_ID_list.astype(jnp.int32))
